# baseline (speedup 1.0000x reference)
import functools
from typing import Any

import jax
import jax.numpy as jnp
from jax import lax
from jax.experimental import pallas as pl
from jax.experimental.pallas import tpu as pltpu
from jax.experimental.pallas import tpu_sc as plsc

N = 2000
D = 128
L = 6
K = 20
C = 10.0
E = N * K

SC_NC = 2
SC_NS = 16
SC_NW = SC_NC * SC_NS
E_PAD = 40960
BPW = E_PAD // SC_NW
CH = 64
NCH = BPW // CH
RING = 4


def _mm_split_body(n1, x_ref, w_ref, b_ref, o1_ref, o2_ref):
    y = (
        jnp.dot(x_ref[...], w_ref[...], preferred_element_type=jnp.float32)
        + b_ref[...]
    )
    o1_ref[...] = y[:, :n1]
    o2_ref[...] = y[:, n1:]


def _mm_split(x, w, b, n1, block_rows):
    m, kdim = x.shape
    n = w.shape[1]
    grid = m // block_rows
    return pl.pallas_call(
        functools.partial(_mm_split_body, n1),
        grid=(grid,),
        in_specs=[
            pl.BlockSpec((block_rows, kdim), lambda i: (i, 0)),
            pl.BlockSpec((kdim, n), lambda i: (0, 0)),
            pl.BlockSpec((1, n), lambda i: (0, 0)),
        ],
        out_specs=[
            pl.BlockSpec((block_rows, n1), lambda i: (i, 0)),
            pl.BlockSpec((block_rows, n - n1), lambda i: (i, 0)),
        ],
        out_shape=[
            jax.ShapeDtypeStruct((m, n1), jnp.float32),
            jax.ShapeDtypeStruct((m, n - n1), jnp.float32),
        ],
    )(x, w, b)


def _sc_gather3(vnt, ewr, tgt_pad, rev_pad):
    mesh = plsc.VectorSubcoreMesh(core_axis_name="c", subcore_axis_name="s")

    @functools.partial(
        pl.kernel, mesh=mesh,
        out_type=[
            jax.ShapeDtypeStruct((E_PAD, 2 * D), jnp.float32),
            jax.ShapeDtypeStruct((E_PAD, D), jnp.float32),
        ],
        scratch_types=(
            [pltpu.VMEM((BPW,), jnp.int32)] * 2
            + [pltpu.VMEM((CH, 2 * D), jnp.float32)] * RING
            + [pltpu.VMEM((CH, D), jnp.float32)] * RING
            + [pltpu.SemaphoreType.DMA] * (4 * RING)
        ),
    )
    def k(vnt_hbm, ewr_hbm, tgt_hbm, rev_hbm, msgvt_out, re_out,
          tgt_v, rev_v, *bufsems):
        abufs = bufsems[:RING]
        bbufs = bufsems[RING:2 * RING]
        gsa = bufsems[2 * RING:2 * RING + RING]
        gsb = bufsems[3 * RING:3 * RING + RING]
        wsa = bufsems[4 * RING:4 * RING + RING]
        wsb = bufsems[5 * RING:]
        wid = lax.axis_index("s") * SC_NC + lax.axis_index("c")
        base = wid * BPW
        pltpu.sync_copy(tgt_hbm.at[pl.ds(base, BPW)], tgt_v)
        pltpu.sync_copy(rev_hbm.at[pl.ds(base, BPW)], rev_v)

        hg, hw = {}, {}

        def start(c):
            s = c % RING
            ii = pl.ds(c * CH, CH)
            hg['a', c] = pltpu.async_copy(
                vnt_hbm.at[tgt_v.at[ii]], abufs[s], gsa[s])
            hg['b', c] = pltpu.async_copy(
                ewr_hbm.at[rev_v.at[ii]], bbufs[s], gsb[s])

        for c in range(RING):
            start(c)
        for c in range(NCH):
            s = c % RING
            oo = pl.ds(base + c * CH, CH)
            hg['a', c].wait()
            hw['a', c] = pltpu.async_copy(abufs[s], msgvt_out.at[oo], wsa[s])
            hg['b', c].wait()
            hw['b', c] = pltpu.async_copy(bbufs[s], re_out.at[oo], wsb[s])
            if c + RING < NCH:
                hw['a', c].wait()
                hw['b', c].wait()
                start(c + RING)
        for c in range(max(0, NCH - RING), NCH):
            hw['a', c].wait()
            hw['b', c].wait()

    return k(vnt, ewr, tgt_pad, rev_pad)


def _n2z(x):
    return jnp.nan_to_num(x, nan=0.0, posinf=0.0, neginf=0.0)


def _asm_body(eao_ref, msgvt_ref, re_ref, rev_ref,
              vws_ref, vwf_ref, rms_ref,
              eup_ref, vup_ref, esum_ref, esq_ref):
    eao = eao_ref[...]
    logits = eao[:, :, :D]
    mx = jnp.max(logits, axis=1, keepdims=True)
    ex = jnp.exp(logits - mx)
    s = jnp.clip(jnp.sum(ex, axis=1, keepdims=True), 1e-10, None)
    attn = ex / (s + 1e-10)
    msgvt = msgvt_ref[...]
    agg = jnp.sum(attn * msgvt[:, :, :D], axis=1, keepdims=True)
    vup_ref[...] = vws_ref[...] + agg

    rev_ok = rev_ref[...] >= 0
    r = jnp.where(rev_ok, re_ref[...], rms_ref[:, 0:1, :])
    e_up = vwf_ref[...] + msgvt[:, :, D:] + eao[:, :, D:] + r
    eup_ref[...] = e_up
    s1 = jnp.sum(e_up, axis=0, keepdims=True)
    s2 = jnp.sum(e_up * e_up, axis=0, keepdims=True)
    esum_ref[...] = jnp.broadcast_to(
        jnp.sum(s1, axis=1, keepdims=True) / 8.0, (1, 8, D))
    esq_ref[...] = jnp.broadcast_to(
        jnp.sum(s2, axis=1, keepdims=True) / 8.0, (1, 8, D))


def _assemble(eao3, msgvt3, re3, rev3, vws3, vwf3, rms3, block_n):
    grid = N // block_n
    spec3w = pl.BlockSpec((block_n, K, 2 * D), lambda i: (i, 0, 0))
    spec3 = pl.BlockSpec((block_n, K, D), lambda i: (i, 0, 0))
    specn = pl.BlockSpec((block_n, 1, D), lambda i: (i, 0, 0))
    return pl.pallas_call(
        _asm_body,
        grid=(grid,),
        in_specs=[
            spec3w, spec3w, spec3,
            pl.BlockSpec((block_n, K, 1), lambda i: (i, 0, 0)),
            specn, specn,
            pl.BlockSpec((1, 8, D), lambda i: (0, 0, 0)),
        ],
        out_specs=[
            spec3,
            specn,
            pl.BlockSpec((1, 8, D), lambda i: (i, 0, 0)),
            pl.BlockSpec((1, 8, D), lambda i: (i, 0, 0)),
        ],
        out_shape=[
            jax.ShapeDtypeStruct((N, K, D), jnp.float32),
            jax.ShapeDtypeStruct((N, 1, D), jnp.float32),
            jax.ShapeDtypeStruct((grid, 8, D), jnp.float32),
            jax.ShapeDtypeStruct((grid, 8, D), jnp.float32),
        ],
    )(eao3, msgvt3, re3, rev3, vws3, vwf3, rms3)


def _vfin_body(v_ref, vup_ref, esum_ref, esq_ref, bnv_ref, bne_ref,
               vnew_ref, escale_ref):
    v_up = vup_ref[...]
    m = jnp.mean(v_up, axis=0, keepdims=True)
    var = jnp.mean((v_up - m) * (v_up - m), axis=0, keepdims=True)
    xn = (v_up - m) / jnp.sqrt(var + 1e-5) * bnv_ref[0:1, :] \
        + bnv_ref[1:2, :]
    vnew_ref[...] = _n2z(v_ref[...] + jnp.maximum(xn, 0.0))

    e_mean = jnp.sum(esum_ref[...], axis=0, keepdims=True) / float(E)
    e_sq = jnp.sum(esq_ref[...], axis=0, keepdims=True) / float(E)
    e_var = e_sq - e_mean * e_mean
    scale = bne_ref[0:1, :] / jnp.sqrt(e_var + 1e-5)
    shift = bne_ref[1:2, :] - e_mean * scale
    escale_ref[...] = jnp.concatenate(
        [scale, shift, jnp.zeros((6, D), jnp.float32)], 0)


def _vfin(v, v_up, esum, esq, bnv, bne):
    g = esum.shape[0]
    return pl.pallas_call(
        _vfin_body,
        in_specs=[
            pl.BlockSpec((N, D), lambda: (0, 0)),
            pl.BlockSpec((N, D), lambda: (0, 0)),
            pl.BlockSpec((g, D), lambda: (0, 0)),
            pl.BlockSpec((g, D), lambda: (0, 0)),
            pl.BlockSpec((2, D), lambda: (0, 0)),
            pl.BlockSpec((2, D), lambda: (0, 0)),
        ],
        out_specs=[
            pl.BlockSpec((N, D), lambda: (0, 0)),
            pl.BlockSpec((8, D), lambda: (0, 0)),
        ],
        out_shape=[
            jax.ShapeDtypeStruct((N, D), jnp.float32),
            jax.ShapeDtypeStruct((8, D), jnp.float32),
        ],
    )(v, v_up, esum, esq, bnv, bne)


def _efin_body(e_ref, eup_ref, sc_ref, o_ref):
    xn = eup_ref[...] * sc_ref[0:1, :] + sc_ref[1:2, :]
    o_ref[...] = _n2z(e_ref[...] + jnp.maximum(xn, 0.0))


def _efin(e, e_up, escale, block_rows):
    grid = E // block_rows
    spec = pl.BlockSpec((block_rows, D), lambda i: (i, 0))
    return pl.pallas_call(
        _efin_body,
        grid=(grid,),
        in_specs=[spec, spec, pl.BlockSpec((8, D), lambda i: (0, 0))],
        out_specs=spec,
        out_shape=jax.ShapeDtypeStruct((E, D), jnp.float32),
    )(e, e_up, escale)


def _encv_body(c_ref, w_ref, b_ref, o_ref):
    o_ref[...] = _n2z(
        jnp.dot(c_ref[...], w_ref[...], preferred_element_type=jnp.float32)
        + b_ref[...]
    )


def _ence_body(d_ref, w_ref, b_ref, o_ref):
    o_ref[...] = _n2z(d_ref[...] * w_ref[...] + b_ref[...])


def _final_e_body(e_ref, w1_ref, b1_ref, w2_ref, b2_ref, wb_ref, bl_ref):
    h = jnp.maximum(
        jnp.dot(e_ref[...], w1_ref[...], preferred_element_type=jnp.float32)
        + b1_ref[...], 0.0)
    e_f = jnp.maximum(
        jnp.dot(h, w2_ref[...], preferred_element_type=jnp.float32)
        + b2_ref[...], 0.0)
    e_f = _n2z(e_f)
    bl_ref[...] = jnp.dot(e_f, wb_ref[...],
                          preferred_element_type=jnp.float32)


def _beta_body(bl_ref, beta_ref):
    bl = bl_ref[...]
    mx = jnp.max(bl, axis=1, keepdims=True)
    lg = jnp.clip(bl - mx, -20.0, 20.0)
    ex = jnp.exp(lg)
    s = jnp.sum(ex, axis=1, keepdims=True)
    beta_ref[...] = ex / (s + 1e-10)


def _final_v_body(v_ref, w1_ref, b1_ref, w2_ref, b2_ref, wp_ref, pi_ref):
    h = jnp.maximum(
        jnp.dot(v_ref[...], w1_ref[...], preferred_element_type=jnp.float32)
        + b1_ref[...], 0.0)
    v_f = jnp.maximum(
        jnp.dot(h, w2_ref[...], preferred_element_type=jnp.float32)
        + b2_ref[...], 0.0)
    v_f = _n2z(v_f)
    pi = C * jnp.tanh(jnp.dot(v_f, wp_ref[...],
                              preferred_element_type=jnp.float32))
    pi_ref[...] = _n2z(pi)


GB_R = 40
CP = 2048


def _topk_body(cb_ref, ct_ref, dist_ref, idx_ref):
    pid = pl.program_id(0)
    cb = cb_ref[...]
    dx = cb[:, 0:1] - ct_ref[0:1, :]
    dy = cb[:, 1:2] - ct_ref[1:2, :]
    dist = jnp.sqrt(jnp.maximum(dx * dx + dy * dy, 0.0))
    rowg = (jax.lax.broadcasted_iota(jnp.int32, (GB_R, CP), 0)
            + pid * GB_R)
    colid = jax.lax.broadcasted_iota(jnp.int32, (GB_R, CP), 1)
    dist = jnp.where((colid == rowg) | (colid >= N), jnp.inf, dist)
    dcols, icols = [], []
    for _ in range(K):
        m = jnp.min(dist, axis=1, keepdims=True)
        am = jnp.min(jnp.where(dist == m, colid, CP), axis=1, keepdims=True)
        dist = jnp.where(colid == am, jnp.inf, dist)
        dcols.append(m)
        icols.append(am)
    dist_ref[...] = jnp.concatenate(dcols, axis=1)
    idx_ref[...] = jnp.concatenate(icols, axis=1)


def _rev_body(idxb_ref, idxt_ref, rev_ref):
    pid = pl.program_id(0)
    rowg = (jax.lax.broadcasted_iota(jnp.int32, (GB_R, CP), 0)
            + pid * GB_R)
    colid = jax.lax.broadcasted_iota(jnp.int32, (GB_R, CP), 1)
    idxb = idxb_ref[...]
    b = jnp.zeros((GB_R, CP), jnp.int32)
    for m in range(K):
        b = b + jnp.where(idxt_ref[m:m + 1, :] == rowg,
                          colid * K + (m + 1), 0)
    rcols = []
    for m in range(K):
        sel = jnp.where(colid == idxb[:, m:m + 1], b, 0)
        rcols.append(jnp.sum(sel, axis=1, keepdims=True))
    rev_ref[...] = jnp.concatenate(rcols, axis=1) - 1


def _build_graph(coords):
    noise = jax.random.normal(jax.random.key(42), coords.shape,
                              dtype=coords.dtype) * 1e-6
    c = coords + noise
    ct = jnp.concatenate([c.T, jnp.zeros((2, CP - N), jnp.float32)], 1)
    dist20, idx20 = pl.pallas_call(
        _topk_body,
        grid=(N // GB_R,),
        in_specs=[
            pl.BlockSpec((GB_R, 2), lambda i: (i, 0)),
            pl.BlockSpec((2, CP), lambda i: (0, 0)),
        ],
        out_specs=[
            pl.BlockSpec((GB_R, K), lambda i: (i, 0)),
            pl.BlockSpec((GB_R, K), lambda i: (i, 0)),
        ],
        out_shape=[
            jax.ShapeDtypeStruct((N, K), jnp.float32),
            jax.ShapeDtypeStruct((N, K), jnp.int32),
        ],
    )(c, ct)
    idxt = jnp.concatenate(
        [idx20.T, jnp.full((K, CP - N), -1, jnp.int32)], 1)
    rev20 = pl.pallas_call(
        _rev_body,
        grid=(N // GB_R,),
        in_specs=[
            pl.BlockSpec((GB_R, K), lambda i: (i, 0)),
            pl.BlockSpec((K, CP), lambda i: (0, 0)),
        ],
        out_specs=pl.BlockSpec((GB_R, K), lambda i: (i, 0)),
        out_shape=jax.ShapeDtypeStruct((N, K), jnp.int32),
    )(idx20, idxt)
    src = jnp.repeat(jnp.arange(N, dtype=jnp.int32), K)
    tgt = idx20.reshape(E)
    edge_dist = dist20.reshape(E)
    return jnp.stack([src, tgt]), edge_dist, rev20.reshape(E)


def kernel(coords, params: dict[str, Any]):
    p = params
    edge_index, edge_dist, rev_idx = _build_graph(coords)
    tgt = edge_index[1]
    rev_clip = jnp.maximum(rev_idx, 0)
    rev3 = rev_idx.reshape(N, K, 1)

    v = pl.pallas_call(
        _encv_body,
        in_specs=[
            pl.BlockSpec((N, 2), lambda: (0, 0)),
            pl.BlockSpec((2, D), lambda: (0, 0)),
            pl.BlockSpec((1, D), lambda: (0, 0)),
        ],
        out_specs=pl.BlockSpec((N, D), lambda: (0, 0)),
        out_shape=jax.ShapeDtypeStruct((N, D), jnp.float32),
    )(coords, p['enc_nW'], p['enc_nb'][None, :])

    eb = 4000
    e = pl.pallas_call(
        _ence_body,
        grid=(E // eb,),
        in_specs=[
            pl.BlockSpec((eb, 1), lambda i: (i, 0)),
            pl.BlockSpec((1, D), lambda i: (0, 0)),
            pl.BlockSpec((1, D), lambda i: (0, 0)),
        ],
        out_specs=pl.BlockSpec((eb, D), lambda i: (i, 0)),
        out_shape=jax.ShapeDtypeStruct((E, D), jnp.float32),
    )(edge_dist[:, None], p['enc_eW'], p['enc_eb'][None, :])

    zeros_d = jnp.zeros((D,), jnp.float32)
    pad_idx = jnp.zeros((E_PAD - E,), jnp.int32)
    tgt_pad = jnp.concatenate([tgt, pad_idx])
    rev_pad = jnp.concatenate([rev_clip, pad_idx])
    for l in range(L):
        vx = jnp.concatenate(
            [v, p['p_vec'][l][None, :], jnp.zeros((7, D), jnp.float32)], 0)
        w_node = jnp.concatenate(
            [p['Wn'][l], p['Wt'][l], p['Ws'][l], p['Wf'][l], p['Wr'][l]], 1)
        b_node = jnp.concatenate(
            [zeros_d, p['Wt_b'][l], p['Ws_b'][l], p['Wf_b'][l],
             p['Wr_b'][l]])[None, :]
        vnt, vrest = _mm_split(vx, w_node, b_node, 2 * D, N + 8)
        vws3 = vrest[:N, 0:D].reshape(N, 1, D)
        vwf3 = vrest[:N, D:2 * D].reshape(N, 1, D)
        rms3 = jnp.tile(vrest[N:N + 1, 2 * D:3 * D], (8, 1)).reshape(1, 8, D)

        w_edge = jnp.concatenate([p['Wa'][l], p['Wo'][l], p['Wr'][l]], 1)
        b_edge = jnp.concatenate(
            [zeros_d, p['Wo_b'][l], p['Wr_b'][l]])[None, :]
        eao, ewr = _mm_split(e, w_edge, b_edge, 2 * D, 2000)

        msgvt_pad, re_pad = _sc_gather3(vnt, ewr, tgt_pad, rev_pad)
        eao3 = eao.reshape(N, K, 2 * D)
        msgvt3 = msgvt_pad.reshape(E_PAD // K, K, 2 * D)
        re3 = re_pad.reshape(E_PAD // K, K, D)

        e_up3, v_up3, esum, esq = _assemble(
            eao3, msgvt3, re3, rev3, vws3, vwf3, rms3, 200)
        bnv = jnp.stack([p['bnv_w'][l], p['bnv_b'][l]])
        bne = jnp.stack([p['bne_w'][l], p['bne_b'][l]])
        g = esum.shape[0]
        v, escale = _vfin(v, v_up3.reshape(N, D),
                          esum.reshape(g * 8, D), esq.reshape(g * 8, D),
                          bnv, bne)
        e = _efin(e, e_up3.reshape(E, D), escale, 4000)

    bl = pl.pallas_call(
        _final_e_body,
        grid=(E // eb,),
        in_specs=[
            pl.BlockSpec((eb, D), lambda i: (i, 0)),
            pl.BlockSpec((D, D), lambda i: (0, 0)),
            pl.BlockSpec((1, D), lambda i: (0, 0)),
            pl.BlockSpec((D, D), lambda i: (0, 0)),
            pl.BlockSpec((1, D), lambda i: (0, 0)),
            pl.BlockSpec((D, 1), lambda i: (0, 0)),
        ],
        out_specs=pl.BlockSpec((eb, 1), lambda i: (i, 0)),
        out_shape=jax.ShapeDtypeStruct((E, 1), jnp.float32),
    )(e, p['edW1'], p['edb1'][None, :], p['edW2'], p['edb2'][None, :],
      p['W_beta'][:, None])

    beta2 = pl.pallas_call(
        _beta_body,
        in_specs=[pl.BlockSpec((N, K), lambda: (0, 0))],
        out_specs=pl.BlockSpec((N, K), lambda: (0, 0)),
        out_shape=jax.ShapeDtypeStruct((N, K), jnp.float32),
    )(bl.reshape(N, K))
    beta = beta2.reshape(E)

    pi2 = pl.pallas_call(
        _final_v_body,
        in_specs=[
            pl.BlockSpec((N, D), lambda: (0, 0)),
            pl.BlockSpec((D, D), lambda: (0, 0)),
            pl.BlockSpec((1, D), lambda: (0, 0)),
            pl.BlockSpec((D, D), lambda: (0, 0)),
            pl.BlockSpec((1, D), lambda: (0, 0)),
            pl.BlockSpec((D, 1), lambda: (0, 0)),
        ],
        out_specs=pl.BlockSpec((N, 1), lambda: (0, 0)),
        out_shape=jax.ShapeDtypeStruct((N, 1), jnp.float32),
    )(v, p['ndW1'], p['ndb1'][None, :], p['ndW2'], p['ndb2'][None, :],
      p['W_pi'][:, None])
    pi = pi2.reshape(N)

    return beta, pi, edge_index, edge_dist


# speedup vs baseline: 1.1721x; 1.1721x over previous
import functools
from typing import Any

import jax
import jax.numpy as jnp
from jax import lax
from jax.experimental import pallas as pl
from jax.experimental.pallas import tpu as pltpu
from jax.experimental.pallas import tpu_sc as plsc

N = 2000
D = 128
L = 6
K = 20
C = 10.0
E = N * K

SC_NC = 2
SC_NS = 16
SC_NW = SC_NC * SC_NS
E_PAD = 40960
BPW = E_PAD // SC_NW
CH = 64
NCH = BPW // CH
RING = 4


def _mm_split_body(n1, x_ref, w_ref, b_ref, o1_ref, o2_ref):
    y = (
        jnp.dot(x_ref[...], w_ref[...], preferred_element_type=jnp.float32)
        + b_ref[...]
    )
    o1_ref[...] = y[:, :n1]
    o2_ref[...] = y[:, n1:]


def _mm_split(x, w, b, n1, block_rows):
    m, kdim = x.shape
    n = w.shape[1]
    grid = m // block_rows
    return pl.pallas_call(
        functools.partial(_mm_split_body, n1),
        grid=(grid,),
        in_specs=[
            pl.BlockSpec((block_rows, kdim), lambda i: (i, 0)),
            pl.BlockSpec((kdim, n), lambda i: (0, 0)),
            pl.BlockSpec((1, n), lambda i: (0, 0)),
        ],
        out_specs=[
            pl.BlockSpec((block_rows, n1), lambda i: (i, 0)),
            pl.BlockSpec((block_rows, n - n1), lambda i: (i, 0)),
        ],
        out_shape=[
            jax.ShapeDtypeStruct((m, n1), jnp.float32),
            jax.ShapeDtypeStruct((m, n - n1), jnp.float32),
        ],
    )(x, w, b)


def _sc_gather(table, idx_pad, width):
    mesh = plsc.VectorSubcoreMesh(core_axis_name="c", subcore_axis_name="s")

    @functools.partial(
        pl.kernel, mesh=mesh,
        out_type=jax.ShapeDtypeStruct((E_PAD, width), jnp.float32),
        scratch_types=(
            [pltpu.VMEM((BPW,), jnp.int32)]
            + [pltpu.VMEM((CH, width), jnp.float32)] * RING
            + [pltpu.SemaphoreType.DMA] * (2 * RING)
        ),
    )
    def k(tab_hbm, idx_hbm, out_hbm, idx_v, *bufsems):
        bufs = bufsems[:RING]
        gs = bufsems[RING:2 * RING]
        ws = bufsems[2 * RING:]
        wid = lax.axis_index("s") * SC_NC + lax.axis_index("c")
        base = wid * BPW
        pltpu.sync_copy(idx_hbm.at[pl.ds(base, BPW)], idx_v)

        hg, hw = {}, {}

        def start(c):
            s = c % RING
            hg[c] = pltpu.async_copy(
                tab_hbm.at[idx_v.at[pl.ds(c * CH, CH)]], bufs[s], gs[s])

        for c in range(RING):
            start(c)
        for c in range(NCH):
            s = c % RING
            hg[c].wait()
            hw[c] = pltpu.async_copy(
                bufs[s], out_hbm.at[pl.ds(base + c * CH, CH)], ws[s])
            if c + RING < NCH:
                hw[c].wait()
                start(c + RING)
        for c in range(max(0, NCH - RING), NCH):
            hw[c].wait()

    return k(table, idx_pad)


def _n2z(x):
    return jnp.nan_to_num(x, nan=0.0, posinf=0.0, neginf=0.0)


def _asm_body(eao_ref, g_ref, erwo_ref, erwr_ref, ewr_ref, rev_ref,
              vws_ref, vwf_ref, vwt_ref, rms_ref,
              eup_ref, euprev_ref, vup_ref, esum_ref, esq_ref):
    eao = eao_ref[...]
    logits = eao[:, :, :D]
    mx = jnp.max(logits, axis=1, keepdims=True)
    ex = jnp.exp(logits - mx)
    s = jnp.clip(jnp.sum(ex, axis=1, keepdims=True), 1e-10, None)
    attn = ex / (s + 1e-10)
    g = g_ref[...]
    agg = jnp.sum(attn * g[:, :, :D], axis=1, keepdims=True)
    vup_ref[...] = vws_ref[...] + agg

    rev_ok = rev_ref[...] >= 0
    r = jnp.where(rev_ok, erwr_ref[...], rms_ref[:, 0:1, :])
    e_up = vwf_ref[...] + g[:, :, D:2 * D] + eao[:, :, D:] + r
    eup_ref[...] = e_up
    euprev_ref[...] = (g[:, :, 2 * D:] + vwt_ref[...] + erwo_ref[...]
                       + ewr_ref[...])
    s1 = jnp.sum(e_up, axis=0, keepdims=True)
    s2 = jnp.sum(e_up * e_up, axis=0, keepdims=True)
    esum_ref[...] = jnp.broadcast_to(
        jnp.sum(s1, axis=1, keepdims=True) / 8.0, (1, 8, D))
    esq_ref[...] = jnp.broadcast_to(
        jnp.sum(s2, axis=1, keepdims=True) / 8.0, (1, 8, D))


def _assemble(eao3, g3, erwo3, erwr3, ewr3, rev3, vws3, vwf3, vwt3, rms3,
              block_n):
    grid = N // block_n
    spec3w = pl.BlockSpec((block_n, K, 2 * D), lambda i: (i, 0, 0))
    spec3g = pl.BlockSpec((block_n, K, 3 * D), lambda i: (i, 0, 0))
    spec3 = pl.BlockSpec((block_n, K, D), lambda i: (i, 0, 0))
    specn = pl.BlockSpec((block_n, 1, D), lambda i: (i, 0, 0))
    return pl.pallas_call(
        _asm_body,
        grid=(grid,),
        in_specs=[
            spec3w, spec3g, spec3, spec3, spec3,
            pl.BlockSpec((block_n, K, 1), lambda i: (i, 0, 0)),
            specn, specn, specn,
            pl.BlockSpec((1, 8, D), lambda i: (0, 0, 0)),
        ],
        out_specs=[
            spec3,
            spec3,
            specn,
            pl.BlockSpec((1, 8, D), lambda i: (i, 0, 0)),
            pl.BlockSpec((1, 8, D), lambda i: (i, 0, 0)),
        ],
        out_shape=[
            jax.ShapeDtypeStruct((N, K, D), jnp.float32),
            jax.ShapeDtypeStruct((N, K, D), jnp.float32),
            jax.ShapeDtypeStruct((N, 1, D), jnp.float32),
            jax.ShapeDtypeStruct((grid, 8, D), jnp.float32),
            jax.ShapeDtypeStruct((grid, 8, D), jnp.float32),
        ],
    )(eao3, g3, erwo3, erwr3, ewr3, rev3, vws3, vwf3, vwt3, rms3)


def _vfin_body(v_ref, vup_ref, esum_ref, esq_ref, bnv_ref, bne_ref,
               vnew_ref, escale_ref):
    v_up = vup_ref[...]
    m = jnp.mean(v_up, axis=0, keepdims=True)
    var = jnp.mean((v_up - m) * (v_up - m), axis=0, keepdims=True)
    xn = (v_up - m) / jnp.sqrt(var + 1e-5) * bnv_ref[0:1, :] \
        + bnv_ref[1:2, :]
    vnew_ref[...] = _n2z(v_ref[...] + jnp.maximum(xn, 0.0))

    e_mean = jnp.sum(esum_ref[...], axis=0, keepdims=True) / float(E)
    e_sq = jnp.sum(esq_ref[...], axis=0, keepdims=True) / float(E)
    e_var = e_sq - e_mean * e_mean
    scale = bne_ref[0:1, :] / jnp.sqrt(e_var + 1e-5)
    shift = bne_ref[1:2, :] - e_mean * scale
    escale_ref[...] = jnp.concatenate(
        [scale, shift, jnp.zeros((6, D), jnp.float32)], 0)


def _vfin(v, v_up, esum, esq, bnv, bne):
    g = esum.shape[0]
    return pl.pallas_call(
        _vfin_body,
        in_specs=[
            pl.BlockSpec((N, D), lambda: (0, 0)),
            pl.BlockSpec((N, D), lambda: (0, 0)),
            pl.BlockSpec((g, D), lambda: (0, 0)),
            pl.BlockSpec((g, D), lambda: (0, 0)),
            pl.BlockSpec((2, D), lambda: (0, 0)),
            pl.BlockSpec((2, D), lambda: (0, 0)),
        ],
        out_specs=[
            pl.BlockSpec((N, D), lambda: (0, 0)),
            pl.BlockSpec((8, D), lambda: (0, 0)),
        ],
        out_shape=[
            jax.ShapeDtypeStruct((N, D), jnp.float32),
            jax.ShapeDtypeStruct((8, D), jnp.float32),
        ],
    )(v, v_up, esum, esq, bnv, bne)


def _efin_body(e_ref, er_ref, eup_ref, euprev_ref, sc_ref, o_ref, orev_ref):
    xn = eup_ref[...] * sc_ref[0:1, :] + sc_ref[1:2, :]
    o_ref[...] = _n2z(e_ref[...] + jnp.maximum(xn, 0.0))
    xr = euprev_ref[...] * sc_ref[0:1, :] + sc_ref[1:2, :]
    orev_ref[...] = _n2z(er_ref[...] + jnp.maximum(xr, 0.0))


def _efin(e, er, e_up, eup_rev, escale, block_rows):
    grid = E // block_rows
    spec = pl.BlockSpec((block_rows, D), lambda i: (i, 0))
    return pl.pallas_call(
        _efin_body,
        grid=(grid,),
        in_specs=[spec, spec, spec, spec,
                  pl.BlockSpec((8, D), lambda i: (0, 0))],
        out_specs=[spec, spec],
        out_shape=[jax.ShapeDtypeStruct((E, D), jnp.float32),
                   jax.ShapeDtypeStruct((E, D), jnp.float32)],
    )(e, er, e_up, eup_rev, escale)


def _encv_body(c_ref, w_ref, b_ref, o_ref):
    o_ref[...] = _n2z(
        jnp.dot(c_ref[...], w_ref[...], preferred_element_type=jnp.float32)
        + b_ref[...]
    )


def _ence_body(d_ref, w_ref, b_ref, o_ref):
    o_ref[...] = _n2z(d_ref[...] * w_ref[...] + b_ref[...])


def _final_e_body(e_ref, w1_ref, b1_ref, w2_ref, b2_ref, wb_ref, bl_ref):
    h = jnp.maximum(
        jnp.dot(e_ref[...], w1_ref[...], preferred_element_type=jnp.float32)
        + b1_ref[...], 0.0)
    e_f = jnp.maximum(
        jnp.dot(h, w2_ref[...], preferred_element_type=jnp.float32)
        + b2_ref[...], 0.0)
    e_f = _n2z(e_f)
    bl_ref[...] = jnp.dot(e_f, wb_ref[...],
                          preferred_element_type=jnp.float32)


def _beta_body(bl_ref, beta_ref):
    bl = bl_ref[...]
    mx = jnp.max(bl, axis=1, keepdims=True)
    lg = jnp.clip(bl - mx, -20.0, 20.0)
    ex = jnp.exp(lg)
    s = jnp.sum(ex, axis=1, keepdims=True)
    beta_ref[...] = ex / (s + 1e-10)


def _final_v_body(v_ref, w1_ref, b1_ref, w2_ref, b2_ref, wp_ref, pi_ref):
    h = jnp.maximum(
        jnp.dot(v_ref[...], w1_ref[...], preferred_element_type=jnp.float32)
        + b1_ref[...], 0.0)
    v_f = jnp.maximum(
        jnp.dot(h, w2_ref[...], preferred_element_type=jnp.float32)
        + b2_ref[...], 0.0)
    v_f = _n2z(v_f)
    pi = C * jnp.tanh(jnp.dot(v_f, wp_ref[...],
                              preferred_element_type=jnp.float32))
    pi_ref[...] = _n2z(pi)


GB_R = 40
CP = 2048


def _topk_body(cb_ref, ct_ref, dist_ref, idx_ref):
    pid = pl.program_id(0)
    cb = cb_ref[...]
    dx = cb[:, 0:1] - ct_ref[0:1, :]
    dy = cb[:, 1:2] - ct_ref[1:2, :]
    dist = jnp.sqrt(jnp.maximum(dx * dx + dy * dy, 0.0))
    rowg = (jax.lax.broadcasted_iota(jnp.int32, (GB_R, CP), 0)
            + pid * GB_R)
    colid = jax.lax.broadcasted_iota(jnp.int32, (GB_R, CP), 1)
    dist = jnp.where((colid == rowg) | (colid >= N), jnp.inf, dist)
    dcols, icols = [], []
    for _ in range(K):
        m = jnp.min(dist, axis=1, keepdims=True)
        am = jnp.min(jnp.where(dist == m, colid, CP), axis=1, keepdims=True)
        dist = jnp.where(colid == am, jnp.inf, dist)
        dcols.append(m)
        icols.append(am)
    dist_ref[...] = jnp.concatenate(dcols, axis=1)
    idx_ref[...] = jnp.concatenate(icols, axis=1)


def _rev_body(idxb_ref, idxt_ref, rev_ref):
    pid = pl.program_id(0)
    rowg = (jax.lax.broadcasted_iota(jnp.int32, (GB_R, CP), 0)
            + pid * GB_R)
    colid = jax.lax.broadcasted_iota(jnp.int32, (GB_R, CP), 1)
    idxb = idxb_ref[...]
    b = jnp.zeros((GB_R, CP), jnp.int32)
    for m in range(K):
        b = b + jnp.where(idxt_ref[m:m + 1, :] == rowg,
                          colid * K + (m + 1), 0)
    rcols = []
    for m in range(K):
        sel = jnp.where(colid == idxb[:, m:m + 1], b, 0)
        rcols.append(jnp.sum(sel, axis=1, keepdims=True))
    rev_ref[...] = jnp.concatenate(rcols, axis=1) - 1


def _build_graph(coords):
    noise = jax.random.normal(jax.random.key(42), coords.shape,
                              dtype=coords.dtype) * 1e-6
    c = coords + noise
    ct = jnp.concatenate([c.T, jnp.zeros((2, CP - N), jnp.float32)], 1)
    dist20, idx20 = pl.pallas_call(
        _topk_body,
        grid=(N // GB_R,),
        in_specs=[
            pl.BlockSpec((GB_R, 2), lambda i: (i, 0)),
            pl.BlockSpec((2, CP), lambda i: (0, 0)),
        ],
        out_specs=[
            pl.BlockSpec((GB_R, K), lambda i: (i, 0)),
            pl.BlockSpec((GB_R, K), lambda i: (i, 0)),
        ],
        out_shape=[
            jax.ShapeDtypeStruct((N, K), jnp.float32),
            jax.ShapeDtypeStruct((N, K), jnp.int32),
        ],
    )(c, ct)
    idxt = jnp.concatenate(
        [idx20.T, jnp.full((K, CP - N), -1, jnp.int32)], 1)
    rev20 = pl.pallas_call(
        _rev_body,
        grid=(N // GB_R,),
        in_specs=[
            pl.BlockSpec((GB_R, K), lambda i: (i, 0)),
            pl.BlockSpec((K, CP), lambda i: (0, 0)),
        ],
        out_specs=pl.BlockSpec((GB_R, K), lambda i: (i, 0)),
        out_shape=jax.ShapeDtypeStruct((N, K), jnp.int32),
    )(idx20, idxt)
    src = jnp.repeat(jnp.arange(N, dtype=jnp.int32), K)
    tgt = idx20.reshape(E)
    edge_dist = dist20.reshape(E)
    return jnp.stack([src, tgt]), edge_dist, rev20.reshape(E)


def kernel(coords, params: dict[str, Any]):
    p = params
    edge_index, edge_dist, rev_idx = _build_graph(coords)
    tgt = edge_index[1]
    rev_clip = jnp.maximum(rev_idx, 0)
    rev3 = rev_idx.reshape(N, K, 1)

    v = pl.pallas_call(
        _encv_body,
        in_specs=[
            pl.BlockSpec((N, 2), lambda: (0, 0)),
            pl.BlockSpec((2, D), lambda: (0, 0)),
            pl.BlockSpec((1, D), lambda: (0, 0)),
        ],
        out_specs=pl.BlockSpec((N, D), lambda: (0, 0)),
        out_shape=jax.ShapeDtypeStruct((N, D), jnp.float32),
    )(coords, p['enc_nW'], p['enc_nb'][None, :])

    eb = 4000
    e = pl.pallas_call(
        _ence_body,
        grid=(E // eb,),
        in_specs=[
            pl.BlockSpec((eb, 1), lambda i: (i, 0)),
            pl.BlockSpec((1, D), lambda i: (0, 0)),
            pl.BlockSpec((1, D), lambda i: (0, 0)),
        ],
        out_specs=pl.BlockSpec((eb, D), lambda i: (i, 0)),
        out_shape=jax.ShapeDtypeStruct((E, D), jnp.float32),
    )(edge_dist[:, None], p['enc_eW'], p['enc_eb'][None, :])

    zeros_d = jnp.zeros((D,), jnp.float32)
    pad_idx = jnp.zeros((E_PAD - E,), jnp.int32)
    tgt_pad = jnp.concatenate([tgt, pad_idx])
    er = e
    for l in range(L):
        vx = jnp.concatenate(
            [v, p['p_vec'][l][None, :], jnp.zeros((7, D), jnp.float32)], 0)
        w_node = jnp.concatenate(
            [p['Wn'][l], p['Wt'][l], p['Wf'][l], p['Ws'][l], p['Wr'][l]], 1)
        b_node = jnp.concatenate(
            [zeros_d, p['Wt_b'][l], p['Wf_b'][l], p['Ws_b'][l],
             p['Wr_b'][l]])[None, :]
        vntf, vrest = _mm_split(vx, w_node, b_node, 3 * D, N + 8)
        vwt3 = vntf[:N, D:2 * D].reshape(N, 1, D)
        vwf3 = vntf[:N, 2 * D:3 * D].reshape(N, 1, D)
        vws3 = vrest[:N, 0:D].reshape(N, 1, D)
        rms3 = jnp.tile(vrest[N:N + 1, D:2 * D], (8, 1)).reshape(1, 8, D)

        w_edge = jnp.concatenate([p['Wa'][l], p['Wo'][l], p['Wr'][l]], 1)
        b_edge = jnp.concatenate(
            [zeros_d, p['Wo_b'][l], p['Wr_b'][l]])[None, :]
        eao, ewr = _mm_split(e, w_edge, b_edge, 2 * D, 2000)
        w_er = jnp.concatenate([p['Wo'][l], p['Wr'][l]], 1)
        b_er = jnp.concatenate([p['Wo_b'][l], p['Wr_b'][l]])[None, :]
        erwo, erwr = _mm_split(er, w_er, b_er, D, 2000)

        g_pad = _sc_gather(vntf, tgt_pad, 3 * D)
        eao3 = eao.reshape(N, K, 2 * D)
        g3 = g_pad.reshape(E_PAD // K, K, 3 * D)

        e_up3, eup_rev3, v_up3, esum, esq = _assemble(
            eao3, g3, erwo.reshape(N, K, D), erwr.reshape(N, K, D),
            ewr.reshape(N, K, D), rev3, vws3, vwf3, vwt3, rms3, 100)
        bnv = jnp.stack([p['bnv_w'][l], p['bnv_b'][l]])
        bne = jnp.stack([p['bne_w'][l], p['bne_b'][l]])
        g = esum.shape[0]
        v, escale = _vfin(v, v_up3.reshape(N, D),
                          esum.reshape(g * 8, D), esq.reshape(g * 8, D),
                          bnv, bne)
        e, er = _efin(e, er, e_up3.reshape(E, D), eup_rev3.reshape(E, D),
                      escale, 4000)

    bl = pl.pallas_call(
        _final_e_body,
        grid=(E // eb,),
        in_specs=[
            pl.BlockSpec((eb, D), lambda i: (i, 0)),
            pl.BlockSpec((D, D), lambda i: (0, 0)),
            pl.BlockSpec((1, D), lambda i: (0, 0)),
            pl.BlockSpec((D, D), lambda i: (0, 0)),
            pl.BlockSpec((1, D), lambda i: (0, 0)),
            pl.BlockSpec((D, 1), lambda i: (0, 0)),
        ],
        out_specs=pl.BlockSpec((eb, 1), lambda i: (i, 0)),
        out_shape=jax.ShapeDtypeStruct((E, 1), jnp.float32),
    )(e, p['edW1'], p['edb1'][None, :], p['edW2'], p['edb2'][None, :],
      p['W_beta'][:, None])

    beta2 = pl.pallas_call(
        _beta_body,
        in_specs=[pl.BlockSpec((N, K), lambda: (0, 0))],
        out_specs=pl.BlockSpec((N, K), lambda: (0, 0)),
        out_shape=jax.ShapeDtypeStruct((N, K), jnp.float32),
    )(bl.reshape(N, K))
    beta = beta2.reshape(E)

    pi2 = pl.pallas_call(
        _final_v_body,
        in_specs=[
            pl.BlockSpec((N, D), lambda: (0, 0)),
            pl.BlockSpec((D, D), lambda: (0, 0)),
            pl.BlockSpec((1, D), lambda: (0, 0)),
            pl.BlockSpec((D, D), lambda: (0, 0)),
            pl.BlockSpec((1, D), lambda: (0, 0)),
            pl.BlockSpec((D, 1), lambda: (0, 0)),
        ],
        out_specs=pl.BlockSpec((N, 1), lambda: (0, 0)),
        out_shape=jax.ShapeDtypeStruct((N, 1), jnp.float32),
    )(v, p['ndW1'], p['ndb1'][None, :], p['ndW2'], p['ndb2'][None, :],
      p['W_pi'][:, None])
    pi = pi2.reshape(N)

    return beta, pi, edge_index, edge_dist


# speedup vs baseline: 1.1731x; 1.0008x over previous
import functools
from typing import Any

import jax
import jax.numpy as jnp
from jax import lax
from jax.experimental import pallas as pl
from jax.experimental.pallas import tpu as pltpu
from jax.experimental.pallas import tpu_sc as plsc

N = 2000
D = 128
L = 6
K = 20
C = 10.0
E = N * K

SC_NC = 2
SC_NS = 16
SC_NW = SC_NC * SC_NS
E_PAD = 40960
BPW = E_PAD // SC_NW
CH = 64
NCH = BPW // CH
RING = 4


def _mm_split_body(n1, x_ref, w_ref, b_ref, o1_ref, o2_ref):
    y = (
        jnp.dot(x_ref[...], w_ref[...], preferred_element_type=jnp.float32)
        + b_ref[...]
    )
    o1_ref[...] = y[:, :n1]
    o2_ref[...] = y[:, n1:]


def _mm_split(x, w, b, n1, block_rows):
    m, kdim = x.shape
    n = w.shape[1]
    grid = m // block_rows
    return pl.pallas_call(
        functools.partial(_mm_split_body, n1),
        grid=(grid,),
        in_specs=[
            pl.BlockSpec((block_rows, kdim), lambda i: (i, 0)),
            pl.BlockSpec((kdim, n), lambda i: (0, 0)),
            pl.BlockSpec((1, n), lambda i: (0, 0)),
        ],
        out_specs=[
            pl.BlockSpec((block_rows, n1), lambda i: (i, 0)),
            pl.BlockSpec((block_rows, n - n1), lambda i: (i, 0)),
        ],
        out_shape=[
            jax.ShapeDtypeStruct((m, n1), jnp.float32),
            jax.ShapeDtypeStruct((m, n - n1), jnp.float32),
        ],
    )(x, w, b)


def _sc_gather(table, idx_pad, width):
    mesh = plsc.VectorSubcoreMesh(core_axis_name="c", subcore_axis_name="s")

    @functools.partial(
        pl.kernel, mesh=mesh,
        out_type=jax.ShapeDtypeStruct((E_PAD, width), jnp.float32),
        scratch_types=(
            [pltpu.VMEM((BPW,), jnp.int32)]
            + [pltpu.VMEM((CH, width), jnp.float32)] * RING
            + [pltpu.SemaphoreType.DMA] * (2 * RING)
        ),
    )
    def k(tab_hbm, idx_hbm, out_hbm, idx_v, *bufsems):
        bufs = bufsems[:RING]
        gs = bufsems[RING:2 * RING]
        ws = bufsems[2 * RING:]
        wid = lax.axis_index("s") * SC_NC + lax.axis_index("c")
        base = wid * BPW
        pltpu.sync_copy(idx_hbm.at[pl.ds(base, BPW)], idx_v)

        hg, hw = {}, {}

        def start(c):
            s = c % RING
            hg[c] = pltpu.async_copy(
                tab_hbm.at[idx_v.at[pl.ds(c * CH, CH)]], bufs[s], gs[s])

        for c in range(RING):
            start(c)
        for c in range(NCH):
            s = c % RING
            hg[c].wait()
            hw[c] = pltpu.async_copy(
                bufs[s], out_hbm.at[pl.ds(base + c * CH, CH)], ws[s])
            if c + RING < NCH:
                hw[c].wait()
                start(c + RING)
        for c in range(max(0, NCH - RING), NCH):
            hw[c].wait()

    return k(table, idx_pad)


def _n2z(x):
    return jnp.nan_to_num(x, nan=0.0, posinf=0.0, neginf=0.0)


def _asm_body(eao_ref, g_ref, erwo_ref, erwr_ref, ewr_ref, rev_ref,
              vws_ref, vwf_ref, vwt_ref, rms_ref,
              eup_ref, euprev_ref, vup_ref, esum_ref, esq_ref):
    eao = eao_ref[...]
    logits = eao[:, :, :D]
    mx = jnp.max(logits, axis=1, keepdims=True)
    ex = jnp.exp(logits - mx)
    s = jnp.clip(jnp.sum(ex, axis=1, keepdims=True), 1e-10, None)
    attn = ex / (s + 1e-10)
    g = g_ref[...]
    agg = jnp.sum(attn * g[:, :, :D], axis=1, keepdims=True)
    vup_ref[...] = vws_ref[...] + agg

    rev_ok = rev_ref[...] >= 0
    r = jnp.where(rev_ok, erwr_ref[...], rms_ref[:, 0:1, :])
    e_up = vwf_ref[...] + g[:, :, D:2 * D] + eao[:, :, D:] + r
    eup_ref[...] = e_up
    euprev_ref[...] = (g[:, :, 2 * D:] + vwt_ref[...] + erwo_ref[...]
                       + ewr_ref[...])
    s1 = jnp.sum(e_up, axis=0, keepdims=True)
    s2 = jnp.sum(e_up * e_up, axis=0, keepdims=True)
    esum_ref[...] = jnp.broadcast_to(
        jnp.sum(s1, axis=1, keepdims=True) / 8.0, (1, 8, D))
    esq_ref[...] = jnp.broadcast_to(
        jnp.sum(s2, axis=1, keepdims=True) / 8.0, (1, 8, D))


def _assemble(eao3, g3, erwo3, erwr3, ewr3, rev3, vws3, vwf3, vwt3, rms3,
              block_n):
    grid = N // block_n
    spec3w = pl.BlockSpec((block_n, K, 2 * D), lambda i: (i, 0, 0))
    spec3g = pl.BlockSpec((block_n, K, 3 * D), lambda i: (i, 0, 0))
    spec3 = pl.BlockSpec((block_n, K, D), lambda i: (i, 0, 0))
    specn = pl.BlockSpec((block_n, 1, D), lambda i: (i, 0, 0))
    return pl.pallas_call(
        _asm_body,
        grid=(grid,),
        in_specs=[
            spec3w, spec3g, spec3, spec3, spec3,
            pl.BlockSpec((block_n, K, 1), lambda i: (i, 0, 0)),
            specn, specn, specn,
            pl.BlockSpec((1, 8, D), lambda i: (0, 0, 0)),
        ],
        out_specs=[
            spec3,
            spec3,
            specn,
            pl.BlockSpec((1, 8, D), lambda i: (i, 0, 0)),
            pl.BlockSpec((1, 8, D), lambda i: (i, 0, 0)),
        ],
        out_shape=[
            jax.ShapeDtypeStruct((N, K, D), jnp.float32),
            jax.ShapeDtypeStruct((N, K, D), jnp.float32),
            jax.ShapeDtypeStruct((N, 1, D), jnp.float32),
            jax.ShapeDtypeStruct((grid, 8, D), jnp.float32),
            jax.ShapeDtypeStruct((grid, 8, D), jnp.float32),
        ],
    )(eao3, g3, erwo3, erwr3, ewr3, rev3, vws3, vwf3, vwt3, rms3)


def _vfin_body(v_ref, vup_ref, esum_ref, esq_ref, bnv_ref, bne_ref,
               vnew_ref, escale_ref):
    v_up = vup_ref[...]
    m = jnp.mean(v_up, axis=0, keepdims=True)
    var = jnp.mean((v_up - m) * (v_up - m), axis=0, keepdims=True)
    xn = (v_up - m) / jnp.sqrt(var + 1e-5) * bnv_ref[0:1, :] \
        + bnv_ref[1:2, :]
    vnew_ref[...] = _n2z(v_ref[...] + jnp.maximum(xn, 0.0))

    e_mean = jnp.sum(esum_ref[...], axis=0, keepdims=True) / float(E)
    e_sq = jnp.sum(esq_ref[...], axis=0, keepdims=True) / float(E)
    e_var = e_sq - e_mean * e_mean
    scale = bne_ref[0:1, :] / jnp.sqrt(e_var + 1e-5)
    shift = bne_ref[1:2, :] - e_mean * scale
    escale_ref[...] = jnp.concatenate(
        [scale, shift, jnp.zeros((6, D), jnp.float32)], 0)


def _vfin(v, v_up, esum, esq, bnv, bne):
    g = esum.shape[0]
    return pl.pallas_call(
        _vfin_body,
        in_specs=[
            pl.BlockSpec((N, D), lambda: (0, 0)),
            pl.BlockSpec((N, D), lambda: (0, 0)),
            pl.BlockSpec((g, D), lambda: (0, 0)),
            pl.BlockSpec((g, D), lambda: (0, 0)),
            pl.BlockSpec((2, D), lambda: (0, 0)),
            pl.BlockSpec((2, D), lambda: (0, 0)),
        ],
        out_specs=[
            pl.BlockSpec((N, D), lambda: (0, 0)),
            pl.BlockSpec((8, D), lambda: (0, 0)),
        ],
        out_shape=[
            jax.ShapeDtypeStruct((N, D), jnp.float32),
            jax.ShapeDtypeStruct((8, D), jnp.float32),
        ],
    )(v, v_up, esum, esq, bnv, bne)


def _efin_body(e_ref, er_ref, eup_ref, euprev_ref, sc_ref, o_ref, orev_ref):
    xn = eup_ref[...] * sc_ref[0:1, :] + sc_ref[1:2, :]
    o_ref[...] = _n2z(e_ref[...] + jnp.maximum(xn, 0.0))
    xr = euprev_ref[...] * sc_ref[0:1, :] + sc_ref[1:2, :]
    orev_ref[...] = _n2z(er_ref[...] + jnp.maximum(xr, 0.0))


def _efin(e, er, e_up, eup_rev, escale, block_rows):
    grid = E // block_rows
    spec = pl.BlockSpec((block_rows, D), lambda i: (i, 0))
    return pl.pallas_call(
        _efin_body,
        grid=(grid,),
        in_specs=[spec, spec, spec, spec,
                  pl.BlockSpec((8, D), lambda i: (0, 0))],
        out_specs=[spec, spec],
        out_shape=[jax.ShapeDtypeStruct((E, D), jnp.float32),
                   jax.ShapeDtypeStruct((E, D), jnp.float32)],
    )(e, er, e_up, eup_rev, escale)


def _encv_body(c_ref, w_ref, b_ref, o_ref):
    o_ref[...] = _n2z(
        jnp.dot(c_ref[...], w_ref[...], preferred_element_type=jnp.float32)
        + b_ref[...]
    )


def _ence_body(d_ref, w_ref, b_ref, o_ref):
    o_ref[...] = _n2z(d_ref[...] * w_ref[...] + b_ref[...])


def _final_e_body(e_ref, w1_ref, b1_ref, w2_ref, b2_ref, wb_ref, bl_ref):
    h = jnp.maximum(
        jnp.dot(e_ref[...], w1_ref[...], preferred_element_type=jnp.float32)
        + b1_ref[...], 0.0)
    e_f = jnp.maximum(
        jnp.dot(h, w2_ref[...], preferred_element_type=jnp.float32)
        + b2_ref[...], 0.0)
    e_f = _n2z(e_f)
    bl_ref[...] = jnp.dot(e_f, wb_ref[...],
                          preferred_element_type=jnp.float32)


def _beta_body(bl_ref, beta_ref):
    bl = bl_ref[...]
    mx = jnp.max(bl, axis=1, keepdims=True)
    lg = jnp.clip(bl - mx, -20.0, 20.0)
    ex = jnp.exp(lg)
    s = jnp.sum(ex, axis=1, keepdims=True)
    beta_ref[...] = ex / (s + 1e-10)


def _final_v_body(v_ref, w1_ref, b1_ref, w2_ref, b2_ref, wp_ref, pi_ref):
    h = jnp.maximum(
        jnp.dot(v_ref[...], w1_ref[...], preferred_element_type=jnp.float32)
        + b1_ref[...], 0.0)
    v_f = jnp.maximum(
        jnp.dot(h, w2_ref[...], preferred_element_type=jnp.float32)
        + b2_ref[...], 0.0)
    v_f = _n2z(v_f)
    pi = C * jnp.tanh(jnp.dot(v_f, wp_ref[...],
                              preferred_element_type=jnp.float32))
    pi_ref[...] = _n2z(pi)


GB_R = 40
CP = 2048


def _topk_body(cb_ref, ct_ref, dist_ref, idx_ref):
    pid = pl.program_id(0)
    cb = cb_ref[...]
    dx = cb[:, 0:1] - ct_ref[0:1, :]
    dy = cb[:, 1:2] - ct_ref[1:2, :]
    dist = jnp.sqrt(jnp.maximum(dx * dx + dy * dy, 0.0))
    rowg = (jax.lax.broadcasted_iota(jnp.int32, (GB_R, CP), 0)
            + pid * GB_R)
    colid = jax.lax.broadcasted_iota(jnp.int32, (GB_R, CP), 1)
    dist = jnp.where((colid == rowg) | (colid >= N), jnp.inf, dist)
    dcols, icols = [], []
    for _ in range(K):
        m = jnp.min(dist, axis=1, keepdims=True)
        am = jnp.min(jnp.where(dist == m, colid, CP), axis=1, keepdims=True)
        dist = jnp.where(colid == am, jnp.inf, dist)
        dcols.append(m)
        icols.append(am)
    dist_ref[...] = jnp.concatenate(dcols, axis=1)
    idx_ref[...] = jnp.concatenate(icols, axis=1)


def _rev_body(idxb_ref, idxt_ref, rev_ref):
    pid = pl.program_id(0)
    rowg = (jax.lax.broadcasted_iota(jnp.int32, (GB_R, CP), 0)
            + pid * GB_R)
    colid = jax.lax.broadcasted_iota(jnp.int32, (GB_R, CP), 1)
    idxb = idxb_ref[...]
    b = jnp.zeros((GB_R, CP), jnp.int32)
    for m in range(K):
        b = b + jnp.where(idxt_ref[m:m + 1, :] == rowg,
                          colid * K + (m + 1), 0)
    rcols = []
    for m in range(K):
        sel = jnp.where(colid == idxb[:, m:m + 1], b, 0)
        rcols.append(jnp.sum(sel, axis=1, keepdims=True))
    rev_ref[...] = jnp.concatenate(rcols, axis=1) - 1


def _build_graph(coords):
    noise = jax.random.normal(jax.random.key(42), coords.shape,
                              dtype=coords.dtype) * 1e-6
    c = coords + noise
    ct = jnp.concatenate([c.T, jnp.zeros((2, CP - N), jnp.float32)], 1)
    dist20, idx20 = pl.pallas_call(
        _topk_body,
        grid=(N // GB_R,),
        in_specs=[
            pl.BlockSpec((GB_R, 2), lambda i: (i, 0)),
            pl.BlockSpec((2, CP), lambda i: (0, 0)),
        ],
        out_specs=[
            pl.BlockSpec((GB_R, K), lambda i: (i, 0)),
            pl.BlockSpec((GB_R, K), lambda i: (i, 0)),
        ],
        out_shape=[
            jax.ShapeDtypeStruct((N, K), jnp.float32),
            jax.ShapeDtypeStruct((N, K), jnp.int32),
        ],
    )(c, ct)
    idxt = jnp.concatenate(
        [idx20.T, jnp.full((K, CP - N), -1, jnp.int32)], 1)
    rev20 = pl.pallas_call(
        _rev_body,
        grid=(N // GB_R,),
        in_specs=[
            pl.BlockSpec((GB_R, K), lambda i: (i, 0)),
            pl.BlockSpec((K, CP), lambda i: (0, 0)),
        ],
        out_specs=pl.BlockSpec((GB_R, K), lambda i: (i, 0)),
        out_shape=jax.ShapeDtypeStruct((N, K), jnp.int32),
    )(idx20, idxt)
    src = jnp.repeat(jnp.arange(N, dtype=jnp.int32), K)
    tgt = idx20.reshape(E)
    edge_dist = dist20.reshape(E)
    return jnp.stack([src, tgt]), edge_dist, rev20.reshape(E)


def kernel(coords, params: dict[str, Any]):
    p = params
    edge_index, edge_dist, rev_idx = _build_graph(coords)
    tgt = edge_index[1]
    rev3 = rev_idx.reshape(N, K, 1)

    v = pl.pallas_call(
        _encv_body,
        in_specs=[
            pl.BlockSpec((N, 2), lambda: (0, 0)),
            pl.BlockSpec((2, D), lambda: (0, 0)),
            pl.BlockSpec((1, D), lambda: (0, 0)),
        ],
        out_specs=pl.BlockSpec((N, D), lambda: (0, 0)),
        out_shape=jax.ShapeDtypeStruct((N, D), jnp.float32),
    )(coords, p['enc_nW'], p['enc_nb'][None, :])

    eb = 4000
    e = pl.pallas_call(
        _ence_body,
        grid=(E // eb,),
        in_specs=[
            pl.BlockSpec((eb, 1), lambda i: (i, 0)),
            pl.BlockSpec((1, D), lambda i: (0, 0)),
            pl.BlockSpec((1, D), lambda i: (0, 0)),
        ],
        out_specs=pl.BlockSpec((eb, D), lambda i: (i, 0)),
        out_shape=jax.ShapeDtypeStruct((E, D), jnp.float32),
    )(edge_dist[:, None], p['enc_eW'], p['enc_eb'][None, :])

    zeros_d = jnp.zeros((D,), jnp.float32)
    pad_idx = jnp.zeros((E_PAD - E,), jnp.int32)
    tgt_pad = jnp.concatenate([tgt, pad_idx])
    er = e
    for l in range(L):
        vx = jnp.concatenate(
            [v, p['p_vec'][l][None, :], jnp.zeros((7, D), jnp.float32)], 0)
        w_node = jnp.concatenate(
            [p['Wn'][l], p['Wt'][l], p['Wf'][l], p['Ws'][l], p['Wr'][l]], 1)
        b_node = jnp.concatenate(
            [zeros_d, p['Wt_b'][l], p['Wf_b'][l], p['Ws_b'][l],
             p['Wr_b'][l]])[None, :]
        vntf, vrest = _mm_split(vx, w_node, b_node, 3 * D, N + 8)
        vwt3 = vntf[:N, D:2 * D].reshape(N, 1, D)
        vwf3 = vntf[:N, 2 * D:3 * D].reshape(N, 1, D)
        vws3 = vrest[:N, 0:D].reshape(N, 1, D)
        rms3 = jnp.tile(vrest[N:N + 1, D:2 * D], (8, 1)).reshape(1, 8, D)

        w_edge = jnp.concatenate([p['Wa'][l], p['Wo'][l], p['Wr'][l]], 1)
        b_edge = jnp.concatenate(
            [zeros_d, p['Wo_b'][l], p['Wr_b'][l]])[None, :]
        eao, ewr = _mm_split(e, w_edge, b_edge, 2 * D, 2000)
        w_er = jnp.concatenate([p['Wo'][l], p['Wr'][l]], 1)
        b_er = jnp.concatenate([p['Wo_b'][l], p['Wr_b'][l]])[None, :]
        erwo, erwr = _mm_split(er, w_er, b_er, D, 2000)

        g_pad = _sc_gather(vntf, tgt_pad, 3 * D)
        eao3 = eao.reshape(N, K, 2 * D)
        g3 = g_pad.reshape(E_PAD // K, K, 3 * D)

        e_up3, eup_rev3, v_up3, esum, esq = _assemble(
            eao3, g3, erwo.reshape(N, K, D), erwr.reshape(N, K, D),
            ewr.reshape(N, K, D), rev3, vws3, vwf3, vwt3, rms3, 100)
        bnv = jnp.stack([p['bnv_w'][l], p['bnv_b'][l]])
        bne = jnp.stack([p['bne_w'][l], p['bne_b'][l]])
        g = esum.shape[0]
        v, escale = _vfin(v, v_up3.reshape(N, D),
                          esum.reshape(g * 8, D), esq.reshape(g * 8, D),
                          bnv, bne)
        e, er = _efin(e, er, e_up3.reshape(E, D), eup_rev3.reshape(E, D),
                      escale, 4000)

    bl = pl.pallas_call(
        _final_e_body,
        grid=(E // eb,),
        in_specs=[
            pl.BlockSpec((eb, D), lambda i: (i, 0)),
            pl.BlockSpec((D, D), lambda i: (0, 0)),
            pl.BlockSpec((1, D), lambda i: (0, 0)),
            pl.BlockSpec((D, D), lambda i: (0, 0)),
            pl.BlockSpec((1, D), lambda i: (0, 0)),
            pl.BlockSpec((D, 1), lambda i: (0, 0)),
        ],
        out_specs=pl.BlockSpec((eb, 1), lambda i: (i, 0)),
        out_shape=jax.ShapeDtypeStruct((E, 1), jnp.float32),
    )(e, p['edW1'], p['edb1'][None, :], p['edW2'], p['edb2'][None, :],
      p['W_beta'][:, None])

    beta2 = pl.pallas_call(
        _beta_body,
        in_specs=[pl.BlockSpec((N, K), lambda: (0, 0))],
        out_specs=pl.BlockSpec((N, K), lambda: (0, 0)),
        out_shape=jax.ShapeDtypeStruct((N, K), jnp.float32),
    )(bl.reshape(N, K))
    beta = beta2.reshape(E)

    pi2 = pl.pallas_call(
        _final_v_body,
        in_specs=[
            pl.BlockSpec((N, D), lambda: (0, 0)),
            pl.BlockSpec((D, D), lambda: (0, 0)),
            pl.BlockSpec((1, D), lambda: (0, 0)),
            pl.BlockSpec((D, D), lambda: (0, 0)),
            pl.BlockSpec((1, D), lambda: (0, 0)),
            pl.BlockSpec((D, 1), lambda: (0, 0)),
        ],
        out_specs=pl.BlockSpec((N, 1), lambda: (0, 0)),
        out_shape=jax.ShapeDtypeStruct((N, 1), jnp.float32),
    )(v, p['ndW1'], p['ndb1'][None, :], p['ndW2'], p['ndb2'][None, :],
      p['W_pi'][:, None])
    pi = pi2.reshape(N)

    return beta, pi, edge_index, edge_dist
